# baseline (speedup 1.0000x reference)
import jax
import jax.numpy as jnp
from jax.experimental import pallas as pl

N = 2048
D_MODEL = 256
OUT_DIM = 32
ALPHA = 0.2
BR = 128


def _proj_kernel(x_ref, w_ref, a_ref, wh_ref, s_ref, t_ref):
    wh = jnp.dot(x_ref[...], w_ref[...], preferred_element_type=jnp.float32)
    wh_ref[...] = wh
    st = jnp.dot(wh, a_ref[...], preferred_element_type=jnp.float32)
    s_ref[...] = st[:, 0:1]
    t_ref[...] = jax.lax.dot_general(
        a_ref[...], wh, (((0,), (1,)), ((), ()))
    )[1:2, :]


def _gat_kernel(adj_ref, wh_ref, s_ref, t_ref, out_ref):
    logits = s_ref[...] + t_ref[...]
    log2e = 1.4426950408889634
    c = jnp.where(logits >= 0.0, -log2e, -ALPHA * log2e)
    e = jnp.exp2(c * logits) * adj_ref[...]
    denom = jnp.sum(e, axis=1, keepdims=True)
    numer = jnp.dot(e, wh_ref[...], preferred_element_type=jnp.float32)
    r = numer / denom
    out_ref[...] = jnp.where(r > 0.0, r, jnp.exp(jnp.minimum(r, 0.0)) - 1.0)


def kernel(input, adj_mat, weights, a_values):
    a_cols = a_values.reshape(2, OUT_DIM).T

    wh, s, t = pl.pallas_call(
        _proj_kernel,
        out_shape=(
            jax.ShapeDtypeStruct((N, OUT_DIM), jnp.float32),
            jax.ShapeDtypeStruct((N, 1), jnp.float32),
            jax.ShapeDtypeStruct((1, N), jnp.float32),
        ),
    )(input, weights, a_cols)

    out = pl.pallas_call(
        _gat_kernel,
        grid=(N // BR,),
        in_specs=[
            pl.BlockSpec((BR, N), lambda i: (i, 0)),
            pl.BlockSpec((N, OUT_DIM), lambda i: (0, 0)),
            pl.BlockSpec((BR, 1), lambda i: (i, 0)),
            pl.BlockSpec((1, N), lambda i: (0, 0)),
        ],
        out_specs=pl.BlockSpec((BR, OUT_DIM), lambda i: (i, 0)),
        out_shape=jax.ShapeDtypeStruct((N, OUT_DIM), jnp.float32),
    )(adj_mat, wh, s, t)
    return out


# speedup vs baseline: 1.2528x; 1.2528x over previous
import jax
import jax.numpy as jnp
from jax.experimental import pallas as pl

N = 2048
D_MODEL = 256
OUT_DIM = 32
ALPHA = 0.2
BR = 512


def _proj_kernel(x_ref, w_ref, a_ref, wh_ref, s_ref, t_ref):
    wh = jnp.dot(x_ref[...], w_ref[...], preferred_element_type=jnp.float32)
    wh_ref[...] = wh
    st = jnp.dot(wh, a_ref[...], preferred_element_type=jnp.float32)
    s_ref[...] = st[:, 0:1]
    t_ref[...] = jax.lax.dot_general(
        a_ref[...], wh, (((0,), (1,)), ((), ()))
    )[1:2, :]


def _gat_kernel(adj_ref, wh_ref, s_ref, t_ref, out_ref):
    logits = s_ref[...] + t_ref[...]
    log2e = 1.4426950408889634
    c = jnp.where(logits >= 0.0, -log2e, -ALPHA * log2e)
    e = jnp.exp2(c * logits) * adj_ref[...]
    denom = jnp.sum(e, axis=1, keepdims=True)
    numer = jnp.dot(e, wh_ref[...], preferred_element_type=jnp.float32)
    r = numer / denom
    out_ref[...] = jnp.where(r > 0.0, r, jnp.exp(jnp.minimum(r, 0.0)) - 1.0)


def kernel(input, adj_mat, weights, a_values):
    a_cols = a_values.reshape(2, OUT_DIM).T

    wh, s, t = pl.pallas_call(
        _proj_kernel,
        out_shape=(
            jax.ShapeDtypeStruct((N, OUT_DIM), jnp.float32),
            jax.ShapeDtypeStruct((N, 1), jnp.float32),
            jax.ShapeDtypeStruct((1, N), jnp.float32),
        ),
    )(input, weights, a_cols)

    out = pl.pallas_call(
        _gat_kernel,
        grid=(N // BR,),
        in_specs=[
            pl.BlockSpec((BR, N), lambda i: (i, 0)),
            pl.BlockSpec((N, OUT_DIM), lambda i: (0, 0)),
            pl.BlockSpec((BR, 1), lambda i: (i, 0)),
            pl.BlockSpec((1, N), lambda i: (0, 0)),
        ],
        out_specs=pl.BlockSpec((BR, OUT_DIM), lambda i: (i, 0)),
        out_shape=jax.ShapeDtypeStruct((N, OUT_DIM), jnp.float32),
    )(adj_mat, wh, s, t)
    return out
